# baseline (device time: 78065 ns/iter reference)
import jax
import jax.numpy as jnp
from jax import lax
from jax.experimental import pallas as pl
from jax.experimental.pallas import tpu as pltpu

N_DEV = 16


def _gelu(y):
    c = 0.7978845608028654
    return 0.5 * y * (1.0 + jnp.tanh(c * (y + 0.044715 * y * y * y)))


def kernel(x, w_mat):
    m_per, k = x.shape
    _, n_per = w_mat.shape

    def body(x_ref, w_ref, out_ref, comm_ref, send_sems, recv_sems):
        my = lax.axis_index("i")
        left = lax.rem(my - 1 + N_DEV, N_DEV)
        right = lax.rem(my + 1, N_DEV)

        barrier_sem = pltpu.get_barrier_semaphore()
        for nbr in (left, right):
            pl.semaphore_signal(
                barrier_sem, inc=1,
                device_id=(nbr,), device_id_type=pl.DeviceIdType.MESH,
            )
        pl.semaphore_wait(barrier_sem, 2)

        comm_ref[0] = x_ref[...]
        out_ref[pl.ds(my * m_per, m_per), :] = _gelu(
            jnp.dot(x_ref[...], w_ref[...], preferred_element_type=jnp.float32)
        )

        for h in range(N_DEV - 1):
            rdma = pltpu.make_async_remote_copy(
                src_ref=comm_ref.at[h],
                dst_ref=comm_ref.at[h + 1],
                send_sem=send_sems.at[h],
                recv_sem=recv_sems.at[h],
                device_id=(right,),
                device_id_type=pl.DeviceIdType.MESH,
            )
            rdma.start()
            rdma.wait()
            origin = lax.rem(my - h - 1 + N_DEV, N_DEV)
            out_ref[pl.ds(origin * m_per, m_per), :] = _gelu(
                jnp.dot(
                    comm_ref[h + 1], w_ref[...],
                    preferred_element_type=jnp.float32,
                )
            )

    return pl.pallas_call(
        body,
        out_shape=jax.ShapeDtypeStruct((N_DEV * m_per, n_per), jnp.float32),
        in_specs=[
            pl.BlockSpec(memory_space=pltpu.VMEM),
            pl.BlockSpec(memory_space=pltpu.VMEM),
        ],
        out_specs=pl.BlockSpec(memory_space=pltpu.VMEM),
        scratch_shapes=[
            pltpu.VMEM((N_DEV, m_per, k), jnp.float32),
            pltpu.SemaphoreType.DMA((N_DEV - 1,)),
            pltpu.SemaphoreType.DMA((N_DEV - 1,)),
        ],
        compiler_params=pltpu.CompilerParams(collective_id=0),
    )(x, w_mat)


# device time: 49118 ns/iter; 1.5893x vs baseline; 1.5893x over previous
import jax
import jax.numpy as jnp
from jax import lax
from jax.experimental import pallas as pl
from jax.experimental.pallas import tpu as pltpu

N_DEV = 16
N_R = 8
N_L = 7

RING = [0, 1, 5, 9, 13, 14, 10, 6, 2, 3, 7, 11, 15, 12, 8, 4]
INV = [RING.index(i) for i in range(N_DEV)]


def _gelu(y):
    c = 0.7978845608028654
    return 0.5 * y * (1.0 + jnp.tanh(c * (y + 0.044715 * y * y * y)))


def kernel(x, w_mat):
    m_per, k = x.shape
    _, n_per = w_mat.shape

    ring = jnp.asarray(RING, jnp.int32)
    inv = jnp.asarray(INV, jnp.int32)
    my = lax.axis_index("i")
    my_r = inv[my]
    right = ring[(my_r + 1) % N_DEV]
    left = ring[(my_r - 1) % N_DEV]
    orig_r = ring[(my_r - 1 - jnp.arange(N_R, dtype=jnp.int32)) % N_DEV]
    orig_l = ring[(my_r + 1 + jnp.arange(N_L, dtype=jnp.int32)) % N_DEV]
    scal = jnp.concatenate(
        [left[None], right[None], orig_r, orig_l]
    ).astype(jnp.int32)

    def body(scal_ref, x_ref, w_ref, out_ref,
             comm_r, comm_l, send_r, recv_r, send_l, recv_l):
        my_id = lax.axis_index("i")
        lft = scal_ref[0]
        rgt = scal_ref[1]

        barrier_sem = pltpu.get_barrier_semaphore()
        for nbr in (lft, rgt):
            pl.semaphore_signal(
                barrier_sem, inc=1,
                device_id=(nbr,), device_id_type=pl.DeviceIdType.MESH,
            )
        pl.semaphore_wait(barrier_sem, 2)

        comm_r[0] = x_ref[...]
        comm_l[0] = x_ref[...]
        out_ref[pl.ds(my_id * m_per, m_per), :] = _gelu(
            jnp.dot(x_ref[...], w_ref[...], preferred_element_type=jnp.float32)
        )

        for h in range(N_R):
            rr = pltpu.make_async_remote_copy(
                src_ref=comm_r.at[h],
                dst_ref=comm_r.at[h + 1],
                send_sem=send_r.at[h],
                recv_sem=recv_r.at[h],
                device_id=(rgt,),
                device_id_type=pl.DeviceIdType.MESH,
            )
            rr.start()
            if h < N_L:
                rl = pltpu.make_async_remote_copy(
                    src_ref=comm_l.at[h],
                    dst_ref=comm_l.at[h + 1],
                    send_sem=send_l.at[h],
                    recv_sem=recv_l.at[h],
                    device_id=(lft,),
                    device_id_type=pl.DeviceIdType.MESH,
                )
                rl.start()
            rr.wait()
            out_ref[pl.ds(scal_ref[2 + h] * m_per, m_per), :] = _gelu(
                jnp.dot(
                    comm_r[h + 1], w_ref[...],
                    preferred_element_type=jnp.float32,
                )
            )
            if h < N_L:
                rl.wait()
                out_ref[pl.ds(scal_ref[2 + N_R + h] * m_per, m_per), :] = _gelu(
                    jnp.dot(
                        comm_l[h + 1], w_ref[...],
                        preferred_element_type=jnp.float32,
                    )
                )

    return pl.pallas_call(
        body,
        out_shape=jax.ShapeDtypeStruct((N_DEV * m_per, n_per), jnp.float32),
        in_specs=[
            pl.BlockSpec(memory_space=pltpu.SMEM),
            pl.BlockSpec(memory_space=pltpu.VMEM),
            pl.BlockSpec(memory_space=pltpu.VMEM),
        ],
        out_specs=pl.BlockSpec(memory_space=pltpu.VMEM),
        scratch_shapes=[
            pltpu.VMEM((N_R + 1, m_per, k), jnp.float32),
            pltpu.VMEM((N_L + 1, m_per, k), jnp.float32),
            pltpu.SemaphoreType.DMA((N_R,)),
            pltpu.SemaphoreType.DMA((N_R,)),
            pltpu.SemaphoreType.DMA((N_L,)),
            pltpu.SemaphoreType.DMA((N_L,)),
        ],
        compiler_params=pltpu.CompilerParams(collective_id=0),
    )(scal, x, w_mat)


# device time: 46149 ns/iter; 1.6916x vs baseline; 1.0643x over previous
import jax
import jax.numpy as jnp
from jax import lax
from jax.experimental import pallas as pl
from jax.experimental.pallas import tpu as pltpu

N_DEV = 16
N_R = 8
N_L = 7

RING = [0, 1, 5, 9, 13, 14, 10, 6, 2, 3, 7, 11, 15, 12, 8, 4]
INV = [RING.index(i) for i in range(N_DEV)]


def _gelu(y):
    c = 0.7978845608028654
    return 0.5 * y * (1.0 + jnp.tanh(c * (y + 0.044715 * y * y * y)))


def kernel(x, w_mat):
    m_per, k = x.shape
    _, n_per = w_mat.shape

    ring = jnp.asarray(RING, jnp.int32)
    inv = jnp.asarray(INV, jnp.int32)
    my = lax.axis_index("i")
    my_r = inv[my]
    right = ring[(my_r + 1) % N_DEV]
    left = ring[(my_r - 1) % N_DEV]
    orig_r = ring[(my_r - 1 - jnp.arange(N_R, dtype=jnp.int32)) % N_DEV]
    orig_l = ring[(my_r + 1 + jnp.arange(N_L, dtype=jnp.int32)) % N_DEV]
    scal = jnp.concatenate(
        [left[None], right[None], orig_r, orig_l]
    ).astype(jnp.int32)

    def body(scal_ref, x_ref, w_ref, out_ref,
             comm_r, comm_l, send_r, recv_r, send_l, recv_l):
        my_id = lax.axis_index("i")
        lft = scal_ref[0]
        rgt = scal_ref[1]

        barrier_sem = pltpu.get_barrier_semaphore()
        for nbr in (lft, rgt):
            pl.semaphore_signal(
                barrier_sem, inc=1,
                device_id=(nbr,), device_id_type=pl.DeviceIdType.MESH,
            )
        pl.semaphore_wait(barrier_sem, 2)

        comm_r[0] = x_ref[...]
        comm_l[0] = x_ref[...]

        rr = [
            pltpu.make_async_remote_copy(
                src_ref=comm_r.at[h],
                dst_ref=comm_r.at[h + 1],
                send_sem=send_r.at[h],
                recv_sem=recv_r.at[h],
                device_id=(rgt,),
                device_id_type=pl.DeviceIdType.MESH,
            )
            for h in range(N_R)
        ]
        rl = [
            pltpu.make_async_remote_copy(
                src_ref=comm_l.at[h],
                dst_ref=comm_l.at[h + 1],
                send_sem=send_l.at[h],
                recv_sem=recv_l.at[h],
                device_id=(lft,),
                device_id_type=pl.DeviceIdType.MESH,
            )
            for h in range(N_L)
        ]

        rr[0].start()
        rl[0].start()
        out_ref[pl.ds(my_id * m_per, m_per), :] = _gelu(
            jnp.dot(x_ref[...], w_ref[...], preferred_element_type=jnp.float32)
        )

        for h in range(N_R):
            rr[h].wait_recv()
            if h + 1 < N_R:
                rr[h + 1].start()
            if h < N_L:
                rl[h].wait_recv()
                if h + 1 < N_L:
                    rl[h + 1].start()
            out_ref[pl.ds(scal_ref[2 + h] * m_per, m_per), :] = _gelu(
                jnp.dot(
                    comm_r[h + 1], w_ref[...],
                    preferred_element_type=jnp.float32,
                )
            )
            if h < N_L:
                out_ref[pl.ds(scal_ref[2 + N_R + h] * m_per, m_per), :] = _gelu(
                    jnp.dot(
                        comm_l[h + 1], w_ref[...],
                        preferred_element_type=jnp.float32,
                    )
                )

        for h in range(N_R):
            rr[h].wait_send()
        for h in range(N_L):
            rl[h].wait_send()

    return pl.pallas_call(
        body,
        out_shape=jax.ShapeDtypeStruct((N_DEV * m_per, n_per), jnp.float32),
        in_specs=[
            pl.BlockSpec(memory_space=pltpu.SMEM),
            pl.BlockSpec(memory_space=pltpu.VMEM),
            pl.BlockSpec(memory_space=pltpu.VMEM),
        ],
        out_specs=pl.BlockSpec(memory_space=pltpu.VMEM),
        scratch_shapes=[
            pltpu.VMEM((N_R + 1, m_per, k), jnp.float32),
            pltpu.VMEM((N_L + 1, m_per, k), jnp.float32),
            pltpu.SemaphoreType.DMA((N_R,)),
            pltpu.SemaphoreType.DMA((N_R,)),
            pltpu.SemaphoreType.DMA((N_L,)),
            pltpu.SemaphoreType.DMA((N_L,)),
        ],
        compiler_params=pltpu.CompilerParams(collective_id=0),
    )(scal, x, w_mat)


# device time: 36762 ns/iter; 2.1235x vs baseline; 1.2553x over previous
import jax
import jax.numpy as jnp
from jax import lax
from jax.experimental import pallas as pl
from jax.experimental.pallas import tpu as pltpu

N_DEV = 16
N_R = 8
N_L = 7
SUB = 4

RING = [0, 1, 5, 9, 13, 14, 10, 6, 2, 3, 7, 11, 15, 12, 8, 4]
INV = [RING.index(i) for i in range(N_DEV)]


def _gelu(y):
    c = 0.7978845608028654
    return 0.5 * y * (1.0 + jnp.tanh(c * (y + 0.044715 * y * y * y)))


def kernel(x, w_mat):
    m_per, k = x.shape
    _, n_per = w_mat.shape
    m_sub = m_per // SUB

    ring = jnp.asarray(RING, jnp.int32)
    inv = jnp.asarray(INV, jnp.int32)
    my = lax.axis_index("i")
    my_r = inv[my]
    right = ring[(my_r + 1) % N_DEV]
    left = ring[(my_r - 1) % N_DEV]
    orig_r = ring[(my_r - 1 - jnp.arange(N_R, dtype=jnp.int32)) % N_DEV]
    orig_l = ring[(my_r + 1 + jnp.arange(N_L, dtype=jnp.int32)) % N_DEV]
    scal = jnp.concatenate(
        [left[None], right[None], orig_r, orig_l]
    ).astype(jnp.int32)

    def body(scal_ref, x_ref, w_ref, out_ref,
             comm_r, comm_l, send_r, recv_r, send_l, recv_l):
        my_id = lax.axis_index("i")
        lft = scal_ref[0]
        rgt = scal_ref[1]

        barrier_sem = pltpu.get_barrier_semaphore()
        for nbr in (lft, rgt):
            pl.semaphore_signal(
                barrier_sem, inc=1,
                device_id=(nbr,), device_id_type=pl.DeviceIdType.MESH,
            )
        pl.semaphore_wait(barrier_sem, 2)

        comm_r[0] = x_ref[...]
        comm_l[0] = x_ref[...]

        def mk(comm, send, recv, h, s, dev):
            return pltpu.make_async_remote_copy(
                src_ref=comm.at[h, pl.ds(s * m_sub, m_sub)],
                dst_ref=comm.at[h + 1, pl.ds(s * m_sub, m_sub)],
                send_sem=send.at[h, s],
                recv_sem=recv.at[h, s],
                device_id=(dev,),
                device_id_type=pl.DeviceIdType.MESH,
            )

        rr = [[mk(comm_r, send_r, recv_r, h, s, rgt) for s in range(SUB)]
              for h in range(N_R)]
        rl = [[mk(comm_l, send_l, recv_l, h, s, lft) for s in range(SUB)]
              for h in range(N_L)]

        for s in range(SUB):
            rr[0][s].start()
            rl[0][s].start()
        out_ref[pl.ds(my_id * m_per, m_per), :] = _gelu(
            jnp.dot(x_ref[...], w_ref[...], preferred_element_type=jnp.float32)
        )

        for h in range(N_R):
            for s in range(SUB):
                rr[h][s].wait_recv()
                if h + 1 < N_R:
                    rr[h + 1][s].start()
            if h < N_L:
                for s in range(SUB):
                    rl[h][s].wait_recv()
                    if h + 1 < N_L:
                        rl[h + 1][s].start()
            out_ref[pl.ds(scal_ref[2 + h] * m_per, m_per), :] = _gelu(
                jnp.dot(
                    comm_r[h + 1], w_ref[...],
                    preferred_element_type=jnp.float32,
                )
            )
            if h < N_L:
                out_ref[pl.ds(scal_ref[2 + N_R + h] * m_per, m_per), :] = _gelu(
                    jnp.dot(
                        comm_l[h + 1], w_ref[...],
                        preferred_element_type=jnp.float32,
                    )
                )

        for h in range(N_R):
            for s in range(SUB):
                rr[h][s].wait_send()
        for h in range(N_L):
            for s in range(SUB):
                rl[h][s].wait_send()

    return pl.pallas_call(
        body,
        out_shape=jax.ShapeDtypeStruct((N_DEV * m_per, n_per), jnp.float32),
        in_specs=[
            pl.BlockSpec(memory_space=pltpu.SMEM),
            pl.BlockSpec(memory_space=pltpu.VMEM),
            pl.BlockSpec(memory_space=pltpu.VMEM),
        ],
        out_specs=pl.BlockSpec(memory_space=pltpu.VMEM),
        scratch_shapes=[
            pltpu.VMEM((N_R + 1, m_per, k), jnp.float32),
            pltpu.VMEM((N_L + 1, m_per, k), jnp.float32),
            pltpu.SemaphoreType.DMA((N_R, SUB)),
            pltpu.SemaphoreType.DMA((N_R, SUB)),
            pltpu.SemaphoreType.DMA((N_L, SUB)),
            pltpu.SemaphoreType.DMA((N_L, SUB)),
        ],
        compiler_params=pltpu.CompilerParams(collective_id=0),
    )(scal, x, w_mat)
